# Initial kernel scaffold; baseline (speedup 1.0000x reference)
import jax
import jax.numpy as jnp
from jax.experimental import pallas as pl


def kernel(node_feats, edge_feats, edge_index, graph_ids, params, Wt, bt):
    raise NotImplementedError("write your pallas kernel here")


# speedup vs baseline: 3.1061x; 3.1061x over previous
import functools

import jax
import jax.numpy as jnp
from jax import lax
from jax.experimental import pallas as pl
from jax.experimental.pallas import tpu as pltpu
from jax.experimental.pallas import tpu_sc as plsc

N = 10000
E = 160000
D = 300
H = D // 2
HP = 160
NUM_LAYERS = 5
G = 64
OUT = 64
NT = 16
NC = 2
EPS = 1e-5

ES_PER_TILE = E // (NC * NT)
ES_CH = 40
ES_STEPS = ES_PER_TILE // ES_CH
AG_PER_TILE = E // NT
AG_CH = 80
AG_STEPS = AG_PER_TILE // AG_CH
ZROWS = 1000

def _sc_edge_scalars_body(dst_hbm, payload_hbm, zrows_hbm, out0, out1,
                          spmem, idx_v, rows_v, sem):
    c = lax.axis_index("c")
    sid = lax.axis_index("s")

    @pl.when(sid < 10)
    def _():
        pltpu.sync_copy(zrows_hbm, spmem.at[pl.ds(sid * ZROWS, ZROWS)])

    plsc.subcore_barrier()

    base = (c * NT + sid) * ES_PER_TILE

    def body(i, carry):
        off = pl.multiple_of(base + i * ES_CH, 8)
        pltpu.sync_copy(dst_hbm.at[pl.ds(off, ES_CH)], idx_v)
        pltpu.async_copy(payload_hbm.at[pl.ds(off, ES_CH)], rows_v, sem).wait()
        pltpu.sync_copy(rows_v, spmem.at[idx_v], add=True)
        return carry

    lax.fori_loop(0, ES_STEPS, body, 0)
    plsc.subcore_barrier()

    @pl.when(jnp.logical_and(c == 0, sid < 10))
    def _():
        sl = pl.ds(sid * ZROWS, ZROWS)
        pltpu.sync_copy(spmem.at[sl], out0.at[sl])

    @pl.when(jnp.logical_and(c == 1, sid < 10))
    def _():
        sl = pl.ds(sid * ZROWS, ZROWS)
        pltpu.sync_copy(spmem.at[sl], out1.at[sl])


def _sc_aggregate_body(t0_hbm, t1_hbm, src_hbm, dst_hbm, zrows_hbm, out0, out1,
                       spmem, sidx_v, didx_v, rows_v, sem):
    c = lax.axis_index("c")
    sid = lax.axis_index("s")

    @pl.when(sid < 10)
    def _():
        pltpu.sync_copy(zrows_hbm, spmem.at[pl.ds(sid * ZROWS, ZROWS)])

    plsc.subcore_barrier()

    base = sid * AG_PER_TILE

    def body_c0(i, carry):
        off = pl.multiple_of(base + i * AG_CH, 8)
        pltpu.sync_copy(src_hbm.at[pl.ds(off, AG_CH)], sidx_v)
        pltpu.sync_copy(dst_hbm.at[pl.ds(off, AG_CH)], didx_v)
        pltpu.async_copy(t0_hbm.at[sidx_v], rows_v, sem).wait()
        pltpu.sync_copy(rows_v, spmem.at[didx_v], add=True)
        return carry

    def body_c1(i, carry):
        off = pl.multiple_of(base + i * AG_CH, 8)
        pltpu.sync_copy(src_hbm.at[pl.ds(off, AG_CH)], sidx_v)
        pltpu.sync_copy(dst_hbm.at[pl.ds(off, AG_CH)], didx_v)
        pltpu.async_copy(t1_hbm.at[sidx_v], rows_v, sem).wait()
        pltpu.sync_copy(rows_v, spmem.at[didx_v], add=True)
        return carry

    @pl.when(c == 0)
    def _():
        lax.fori_loop(0, AG_STEPS, body_c0, 0)

    @pl.when(c == 1)
    def _():
        lax.fori_loop(0, AG_STEPS, body_c1, 0)

    plsc.subcore_barrier()

    @pl.when(jnp.logical_and(c == 0, sid < 10))
    def _():
        sl = pl.ds(sid * ZROWS, ZROWS)
        pltpu.sync_copy(spmem.at[sl], out0.at[sl])

    @pl.when(jnp.logical_and(c == 1, sid < 10))
    def _():
        sl = pl.ds(sid * ZROWS, ZROWS)
        pltpu.sync_copy(spmem.at[sl], out1.at[sl])


@functools.cache
def _build_sc_kernels():
    mesh = plsc.VectorSubcoreMesh(core_axis_name="c", subcore_axis_name="s",
                                  num_cores=NC, num_subcores=NT)
    sc_edge_scalars = pl.kernel(
        _sc_edge_scalars_body,
        out_type=[
            jax.ShapeDtypeStruct((N, 16), jnp.float32),
            jax.ShapeDtypeStruct((N, 16), jnp.float32),
        ],
        mesh=mesh,
        scratch_types=[
            pltpu.VMEM_SHARED((N, 16), jnp.float32),
            pltpu.VMEM((ES_CH,), jnp.int32),
            pltpu.VMEM((ES_CH, 16), jnp.float32),
            pltpu.SemaphoreType.DMA,
        ],
        compiler_params=pltpu.CompilerParams(use_tc_tiling_on_sc=False),
    )
    sc_aggregate = pl.kernel(
        _sc_aggregate_body,
        out_type=[
            jax.ShapeDtypeStruct((N, HP), jnp.float32),
            jax.ShapeDtypeStruct((N, HP), jnp.float32),
        ],
        mesh=mesh,
        scratch_types=[
            pltpu.VMEM_SHARED((N, HP), jnp.float32),
            pltpu.VMEM((AG_CH,), jnp.int32),
            pltpu.VMEM((AG_CH,), jnp.int32),
            pltpu.VMEM((AG_CH, HP), jnp.float32),
            pltpu.SemaphoreType.DMA,
        ],
        compiler_params=pltpu.CompilerParams(use_tc_tiling_on_sc=False),
    )
    return sc_edge_scalars, sc_aggregate


RB = 1000
NRB = N // RB


def _tc_mlp_body(a0_ref, a1_ref, es0_ref, es1_ref, stats_ref, gamma_ref,
                 beta_ref, w1_ref, b1_ref, w2_ref, b2_ref,
                 z0_ref, z1_ref, ostats_ref):
    i = pl.program_id(0)
    mean = stats_ref[0:1, :] * (1.0 / N)
    var = stats_ref[1:2, :] * (1.0 / N) - mean * mean
    s = gamma_ref[...] * lax.rsqrt(var + EPS)
    t = beta_ref[...] - mean * s

    esum = es0_ref[:, 0:1] + es1_ref[:, 0:1]
    indeg = es0_ref[:, 1:2] + es1_ref[:, 1:2]

    a = jnp.concatenate([a0_ref[...][:, :H], a1_ref[...][:, :H]], axis=1)
    agg = a * s + indeg * t + esum
    pre = jnp.dot(agg, w1_ref[...], preferred_element_type=jnp.float32)
    pre += b1_ref[...]
    h1 = jnp.maximum(pre, 0.0)
    z = jnp.dot(h1, w2_ref[...], preferred_element_type=jnp.float32)
    z += b2_ref[...]

    zpad = jnp.zeros((RB, HP - H), jnp.float32)
    z0_ref[...] = jnp.concatenate([z[:, :H], zpad], axis=1)
    z1_ref[...] = jnp.concatenate([z[:, H:], zpad], axis=1)

    @pl.when(i == 0)
    def _():
        ostats_ref[...] = jnp.zeros_like(ostats_ref)

    ostats_ref[0:1, :] += jnp.sum(z, axis=0, keepdims=True)
    ostats_ref[1:2, :] += jnp.sum(z * z, axis=0, keepdims=True)


_tc_mlp = pl.pallas_call(
    _tc_mlp_body,
    grid=(NRB,),
    in_specs=[
        pl.BlockSpec((RB, HP), lambda i: (i, 0)),
        pl.BlockSpec((RB, HP), lambda i: (i, 0)),
        pl.BlockSpec((RB, 16), lambda i: (i, 0)),
        pl.BlockSpec((RB, 16), lambda i: (i, 0)),
        pl.BlockSpec((2, D), lambda i: (0, 0)),
        pl.BlockSpec((1, D), lambda i: (0, 0)),
        pl.BlockSpec((1, D), lambda i: (0, 0)),
        pl.BlockSpec((D, 2 * D), lambda i: (0, 0)),
        pl.BlockSpec((1, 2 * D), lambda i: (0, 0)),
        pl.BlockSpec((2 * D, D), lambda i: (0, 0)),
        pl.BlockSpec((1, D), lambda i: (0, 0)),
    ],
    out_specs=[
        pl.BlockSpec((RB, HP), lambda i: (i, 0)),
        pl.BlockSpec((RB, HP), lambda i: (i, 0)),
        pl.BlockSpec((2, D), lambda i: (0, 0)),
    ],
    out_shape=[
        jax.ShapeDtypeStruct((N, HP), jnp.float32),
        jax.ShapeDtypeStruct((N, HP), jnp.float32),
        jax.ShapeDtypeStruct((2, D), jnp.float32),
    ],
)


def _tc_readout_body(z0_ref, z1_ref, gids_ref, stats_ref, gamma_ref, beta_ref,
                     wt_ref, bt_ref, out_ref, acc0, acc1, cnt):
    i = pl.program_id(0)

    @pl.when(i == 0)
    def _():
        acc0[...] = jnp.zeros_like(acc0)
        acc1[...] = jnp.zeros_like(acc1)
        cnt[...] = jnp.zeros_like(cnt)

    gids = gids_ref[...]
    giota = lax.broadcasted_iota(jnp.int32, (RB, G), 1).astype(jnp.float32)
    m = (giota == gids).astype(jnp.float32)
    dn = (((0,), (0,)), ((), ()))
    acc0[...] += lax.dot_general(m, z0_ref[...][:, :H], dn,
                                 preferred_element_type=jnp.float32,
                  precision=lax.Precision.HIGHEST)
    acc1[...] += lax.dot_general(m, z1_ref[...][:, :H], dn,
                                 preferred_element_type=jnp.float32,
                  precision=lax.Precision.HIGHEST)
    ones = jnp.ones((RB, 1), jnp.float32)
    cnt[...] += lax.dot_general(m, ones, dn,
                                preferred_element_type=jnp.float32,
                  precision=lax.Precision.HIGHEST)

    @pl.when(i == NRB - 1)
    def _():
        mean = stats_ref[0:1, :] * (1.0 / N)
        var = stats_ref[1:2, :] * (1.0 / N) - mean * mean
        s = gamma_ref[...] * lax.rsqrt(var + EPS)
        t = beta_ref[...] - mean * s
        c = cnt[...]
        inv = 1.0 / jnp.maximum(c, 1.0)
        g0 = (acc0[...] * s[:, :H] + c * t[:, :H]) * inv
        g1 = (acc1[...] * s[:, H:] + c * t[:, H:]) * inv
        gm = jnp.concatenate([g0, g1], axis=1)
        o = jnp.dot(gm, wt_ref[...], preferred_element_type=jnp.float32)
        out_ref[...] = o + bt_ref[...]


_tc_readout = pl.pallas_call(
    _tc_readout_body,
    grid=(NRB,),
    in_specs=[
        pl.BlockSpec((RB, HP), lambda i: (i, 0)),
        pl.BlockSpec((RB, HP), lambda i: (i, 0)),
        pl.BlockSpec((RB, 1), lambda i: (i, 0)),
        pl.BlockSpec((2, D), lambda i: (0, 0)),
        pl.BlockSpec((1, D), lambda i: (0, 0)),
        pl.BlockSpec((1, D), lambda i: (0, 0)),
        pl.BlockSpec((D, OUT), lambda i: (0, 0)),
        pl.BlockSpec((1, OUT), lambda i: (0, 0)),
    ],
    out_specs=pl.BlockSpec((G, OUT), lambda i: (0, 0)),
    out_shape=jax.ShapeDtypeStruct((G, OUT), jnp.float32),
    scratch_shapes=[
        pltpu.VMEM((G, H), jnp.float32),
        pltpu.VMEM((G, H), jnp.float32),
        pltpu.VMEM((G, 1), jnp.float32),
    ],
)


def kernel(node_feats, edge_feats, edge_index, graph_ids, params, Wt, bt):
    src = edge_index[0]
    dst = edge_index[1]

    nf0 = jnp.pad(node_feats[:, :H], ((0, 0), (0, HP - H)))
    nf1 = jnp.pad(node_feats[:, H:], ((0, 0), (0, HP - H)))
    ones_e = jnp.ones((E,), jnp.float32)
    payload = jnp.tile(jnp.stack([edge_feats[:, 0], ones_e], axis=-1), (1, 8))
    zrows16 = jnp.zeros((ZROWS, 16), jnp.float32)
    zrowsH = jnp.zeros((ZROWS, HP), jnp.float32)
    gids = graph_ids.astype(jnp.float32).reshape(N, 1)

    sc_edge_scalars, sc_aggregate = _build_sc_kernels()
    es0, es1 = sc_edge_scalars(dst, payload, zrows16)

    stats = jnp.stack([jnp.zeros((D,), jnp.float32),
                       jnp.full((D,), N * (1.0 - EPS), jnp.float32)])
    gamma_p = jnp.ones((1, D), jnp.float32)
    beta_p = jnp.zeros((1, D), jnp.float32)

    h0, h1 = nf0, nf1
    for (W1, b1, W2, b2, gamma, beta) in params:
        a0, a1 = sc_aggregate(h0, h1, src, dst, zrowsH)
        h0, h1, stats = _tc_mlp(a0, a1, es0, es1, stats, gamma_p, beta_p,
                                W1, b1.reshape(1, 2 * D), W2,
                                b2.reshape(1, D))
        gamma_p = gamma.reshape(1, D)
        beta_p = beta.reshape(1, D)

    return _tc_readout(h0, h1, gids, stats, gamma_p, beta_p,
                       Wt, bt.reshape(1, OUT))


# speedup vs baseline: 4.3844x; 1.4116x over previous
import functools

import jax
import jax.numpy as jnp
from jax import lax
from jax.experimental import pallas as pl
from jax.experimental.pallas import tpu as pltpu
from jax.experimental.pallas import tpu_sc as plsc

N = 10000
E = 160000
D = 300
H = D // 2
HP = 160
NUM_LAYERS = 5
G = 64
OUT = 64
NT = 16
NC = 2
EPS = 1e-5

ES_PER_TILE = E // (NC * NT)
ES_CH = 40
ES_STEPS = ES_PER_TILE // ES_CH
AG_PER_TILE = E // NT
AG_CH = 80
AG_STEPS = AG_PER_TILE // AG_CH
ZROWS = 1000

def _sc_edge_scalars_body(dst_hbm, payload_hbm, zrows_hbm, out0, out1,
                          spmem, idx_v, rows_v, sem):
    c = lax.axis_index("c")
    sid = lax.axis_index("s")

    @pl.when(sid < 10)
    def _():
        pltpu.sync_copy(zrows_hbm, spmem.at[pl.ds(sid * ZROWS, ZROWS)])

    plsc.subcore_barrier()

    base = (c * NT + sid) * ES_PER_TILE

    def body(i, carry):
        off = pl.multiple_of(base + i * ES_CH, 8)
        pltpu.sync_copy(dst_hbm.at[pl.ds(off, ES_CH)], idx_v)
        pltpu.async_copy(payload_hbm.at[pl.ds(off, ES_CH)], rows_v, sem).wait()
        pltpu.sync_copy(rows_v, spmem.at[idx_v], add=True)
        return carry

    lax.fori_loop(0, ES_STEPS, body, 0)
    plsc.subcore_barrier()

    @pl.when(jnp.logical_and(c == 0, sid < 10))
    def _():
        sl = pl.ds(sid * ZROWS, ZROWS)
        pltpu.sync_copy(spmem.at[sl], out0.at[sl])

    @pl.when(jnp.logical_and(c == 1, sid < 10))
    def _():
        sl = pl.ds(sid * ZROWS, ZROWS)
        pltpu.sync_copy(spmem.at[sl], out1.at[sl])


def _sc_aggregate_body(t0_hbm, t1_hbm, src_hbm, dst_hbm, zrows_hbm, out0, out1,
                       spmem, sidx_a, sidx_b, didx_v, rows_a, rows_b,
                       sem_a, sem_b):
    c = lax.axis_index("c")
    sid = lax.axis_index("s")

    @pl.when(sid < 10)
    def _():
        pltpu.sync_copy(zrows_hbm, spmem.at[pl.ds(sid * ZROWS, ZROWS)])

    plsc.subcore_barrier()

    base = sid * AG_PER_TILE

    def run(tbl):
        def start(i, sidx, rows, sem):
            off = pl.multiple_of(base + i * AG_CH, 8)
            pltpu.sync_copy(src_hbm.at[pl.ds(off, AG_CH)], sidx)
            pltpu.async_copy(tbl.at[sidx], rows, sem)

        def finish(i, sidx, rows, sem):
            off = pl.multiple_of(base + i * AG_CH, 8)
            pltpu.sync_copy(dst_hbm.at[pl.ds(off, AG_CH)], didx_v)
            pltpu.make_async_copy(tbl.at[sidx], rows, sem).wait()
            pltpu.sync_copy(rows, spmem.at[didx_v], add=True)

        start(0, sidx_a, rows_a, sem_a)

        def body(i, carry):
            @pl.when(i % 2 == 0)
            def _():
                @pl.when(i + 1 < AG_STEPS)
                def _():
                    start(i + 1, sidx_b, rows_b, sem_b)
                finish(i, sidx_a, rows_a, sem_a)

            @pl.when(i % 2 == 1)
            def _():
                @pl.when(i + 1 < AG_STEPS)
                def _():
                    start(i + 1, sidx_a, rows_a, sem_a)
                finish(i, sidx_b, rows_b, sem_b)

            return carry

        lax.fori_loop(0, AG_STEPS, body, 0)

    @pl.when(c == 0)
    def _():
        run(t0_hbm)

    @pl.when(c == 1)
    def _():
        run(t1_hbm)

    plsc.subcore_barrier()

    @pl.when(jnp.logical_and(c == 0, sid < 10))
    def _():
        sl = pl.ds(sid * ZROWS, ZROWS)
        pltpu.sync_copy(spmem.at[sl], out0.at[sl])

    @pl.when(jnp.logical_and(c == 1, sid < 10))
    def _():
        sl = pl.ds(sid * ZROWS, ZROWS)
        pltpu.sync_copy(spmem.at[sl], out1.at[sl])


@functools.cache
def _build_sc_kernels():
    mesh = plsc.VectorSubcoreMesh(core_axis_name="c", subcore_axis_name="s",
                                  num_cores=NC, num_subcores=NT)
    sc_edge_scalars = pl.kernel(
        _sc_edge_scalars_body,
        out_type=[
            jax.ShapeDtypeStruct((N, 16), jnp.float32),
            jax.ShapeDtypeStruct((N, 16), jnp.float32),
        ],
        mesh=mesh,
        scratch_types=[
            pltpu.VMEM_SHARED((N, 16), jnp.float32),
            pltpu.VMEM((ES_CH,), jnp.int32),
            pltpu.VMEM((ES_CH, 16), jnp.float32),
            pltpu.SemaphoreType.DMA,
        ],
        compiler_params=pltpu.CompilerParams(use_tc_tiling_on_sc=False),
    )
    sc_aggregate = pl.kernel(
        _sc_aggregate_body,
        out_type=[
            jax.ShapeDtypeStruct((N, HP), jnp.float32),
            jax.ShapeDtypeStruct((N, HP), jnp.float32),
        ],
        mesh=mesh,
        scratch_types=[
            pltpu.VMEM_SHARED((N, HP), jnp.float32),
            pltpu.VMEM((AG_CH,), jnp.int32),
            pltpu.VMEM((AG_CH,), jnp.int32),
            pltpu.VMEM((AG_CH,), jnp.int32),
            pltpu.VMEM((AG_CH, HP), jnp.float32),
            pltpu.VMEM((AG_CH, HP), jnp.float32),
            pltpu.SemaphoreType.DMA,
            pltpu.SemaphoreType.DMA,
        ],
        compiler_params=pltpu.CompilerParams(use_tc_tiling_on_sc=False),
    )
    return sc_edge_scalars, sc_aggregate


RB = 1000
NRB = N // RB


def _tc_mlp_body(a0_ref, a1_ref, es0_ref, es1_ref, stats_ref, gamma_ref,
                 beta_ref, w1_ref, b1_ref, w2_ref, b2_ref,
                 z0_ref, z1_ref, ostats_ref):
    i = pl.program_id(0)
    mean = stats_ref[0:1, :] * (1.0 / N)
    var = stats_ref[1:2, :] * (1.0 / N) - mean * mean
    s = gamma_ref[...] * lax.rsqrt(var + EPS)
    t = beta_ref[...] - mean * s

    esum = es0_ref[:, 0:1] + es1_ref[:, 0:1]
    indeg = es0_ref[:, 1:2] + es1_ref[:, 1:2]

    a = jnp.concatenate([a0_ref[...][:, :H], a1_ref[...][:, :H]], axis=1)
    agg = a * s + indeg * t + esum
    pre = jnp.dot(agg, w1_ref[...], preferred_element_type=jnp.float32)
    pre += b1_ref[...]
    h1 = jnp.maximum(pre, 0.0)
    z = jnp.dot(h1, w2_ref[...], preferred_element_type=jnp.float32)
    z += b2_ref[...]

    zpad = jnp.zeros((RB, HP - H), jnp.float32)
    z0_ref[...] = jnp.concatenate([z[:, :H], zpad], axis=1)
    z1_ref[...] = jnp.concatenate([z[:, H:], zpad], axis=1)

    @pl.when(i == 0)
    def _():
        ostats_ref[...] = jnp.zeros_like(ostats_ref)

    ostats_ref[0:1, :] += jnp.sum(z, axis=0, keepdims=True)
    ostats_ref[1:2, :] += jnp.sum(z * z, axis=0, keepdims=True)


_tc_mlp = pl.pallas_call(
    _tc_mlp_body,
    grid=(NRB,),
    in_specs=[
        pl.BlockSpec((RB, HP), lambda i: (i, 0)),
        pl.BlockSpec((RB, HP), lambda i: (i, 0)),
        pl.BlockSpec((RB, 16), lambda i: (i, 0)),
        pl.BlockSpec((RB, 16), lambda i: (i, 0)),
        pl.BlockSpec((2, D), lambda i: (0, 0)),
        pl.BlockSpec((1, D), lambda i: (0, 0)),
        pl.BlockSpec((1, D), lambda i: (0, 0)),
        pl.BlockSpec((D, 2 * D), lambda i: (0, 0)),
        pl.BlockSpec((1, 2 * D), lambda i: (0, 0)),
        pl.BlockSpec((2 * D, D), lambda i: (0, 0)),
        pl.BlockSpec((1, D), lambda i: (0, 0)),
    ],
    out_specs=[
        pl.BlockSpec((RB, HP), lambda i: (i, 0)),
        pl.BlockSpec((RB, HP), lambda i: (i, 0)),
        pl.BlockSpec((2, D), lambda i: (0, 0)),
    ],
    out_shape=[
        jax.ShapeDtypeStruct((N, HP), jnp.float32),
        jax.ShapeDtypeStruct((N, HP), jnp.float32),
        jax.ShapeDtypeStruct((2, D), jnp.float32),
    ],
)


def _tc_readout_body(z0_ref, z1_ref, gids_ref, stats_ref, gamma_ref, beta_ref,
                     wt_ref, bt_ref, out_ref, acc0, acc1, cnt):
    i = pl.program_id(0)

    @pl.when(i == 0)
    def _():
        acc0[...] = jnp.zeros_like(acc0)
        acc1[...] = jnp.zeros_like(acc1)
        cnt[...] = jnp.zeros_like(cnt)

    gids = gids_ref[...]
    giota = lax.broadcasted_iota(jnp.int32, (RB, G), 1).astype(jnp.float32)
    m = (giota == gids).astype(jnp.float32)
    dn = (((0,), (0,)), ((), ()))
    acc0[...] += lax.dot_general(m, z0_ref[...][:, :H], dn,
                                 preferred_element_type=jnp.float32,
                  precision=lax.Precision.HIGHEST)
    acc1[...] += lax.dot_general(m, z1_ref[...][:, :H], dn,
                                 preferred_element_type=jnp.float32,
                  precision=lax.Precision.HIGHEST)
    ones = jnp.ones((RB, 1), jnp.float32)
    cnt[...] += lax.dot_general(m, ones, dn,
                                preferred_element_type=jnp.float32,
                  precision=lax.Precision.HIGHEST)

    @pl.when(i == NRB - 1)
    def _():
        mean = stats_ref[0:1, :] * (1.0 / N)
        var = stats_ref[1:2, :] * (1.0 / N) - mean * mean
        s = gamma_ref[...] * lax.rsqrt(var + EPS)
        t = beta_ref[...] - mean * s
        c = cnt[...]
        inv = 1.0 / jnp.maximum(c, 1.0)
        g0 = (acc0[...] * s[:, :H] + c * t[:, :H]) * inv
        g1 = (acc1[...] * s[:, H:] + c * t[:, H:]) * inv
        gm = jnp.concatenate([g0, g1], axis=1)
        o = jnp.dot(gm, wt_ref[...], preferred_element_type=jnp.float32)
        out_ref[...] = o + bt_ref[...]


_tc_readout = pl.pallas_call(
    _tc_readout_body,
    grid=(NRB,),
    in_specs=[
        pl.BlockSpec((RB, HP), lambda i: (i, 0)),
        pl.BlockSpec((RB, HP), lambda i: (i, 0)),
        pl.BlockSpec((RB, 1), lambda i: (i, 0)),
        pl.BlockSpec((2, D), lambda i: (0, 0)),
        pl.BlockSpec((1, D), lambda i: (0, 0)),
        pl.BlockSpec((1, D), lambda i: (0, 0)),
        pl.BlockSpec((D, OUT), lambda i: (0, 0)),
        pl.BlockSpec((1, OUT), lambda i: (0, 0)),
    ],
    out_specs=pl.BlockSpec((G, OUT), lambda i: (0, 0)),
    out_shape=jax.ShapeDtypeStruct((G, OUT), jnp.float32),
    scratch_shapes=[
        pltpu.VMEM((G, H), jnp.float32),
        pltpu.VMEM((G, H), jnp.float32),
        pltpu.VMEM((G, 1), jnp.float32),
    ],
)


def kernel(node_feats, edge_feats, edge_index, graph_ids, params, Wt, bt):
    src = edge_index[0]
    dst = edge_index[1]

    nf0 = jnp.pad(node_feats[:, :H], ((0, 0), (0, HP - H)))
    nf1 = jnp.pad(node_feats[:, H:], ((0, 0), (0, HP - H)))
    ones_e = jnp.ones((E,), jnp.float32)
    payload = jnp.tile(jnp.stack([edge_feats[:, 0], ones_e], axis=-1), (1, 8))
    zrows16 = jnp.zeros((ZROWS, 16), jnp.float32)
    zrowsH = jnp.zeros((ZROWS, HP), jnp.float32)
    gids = graph_ids.astype(jnp.float32).reshape(N, 1)

    sc_edge_scalars, sc_aggregate = _build_sc_kernels()
    es0, es1 = sc_edge_scalars(dst, payload, zrows16)

    stats = jnp.stack([jnp.zeros((D,), jnp.float32),
                       jnp.full((D,), N * (1.0 - EPS), jnp.float32)])
    gamma_p = jnp.ones((1, D), jnp.float32)
    beta_p = jnp.zeros((1, D), jnp.float32)

    h0, h1 = nf0, nf1
    for (W1, b1, W2, b2, gamma, beta) in params:
        a0, a1 = sc_aggregate(h0, h1, src, dst, zrowsH)
        h0, h1, stats = _tc_mlp(a0, a1, es0, es1, stats, gamma_p, beta_p,
                                W1, b1.reshape(1, 2 * D), W2,
                                b2.reshape(1, D))
        gamma_p = gamma.reshape(1, D)
        beta_p = beta.reshape(1, D)

    return _tc_readout(h0, h1, gids, stats, gamma_p, beta_p,
                       Wt, bt.reshape(1, OUT))


# speedup vs baseline: 4.9598x; 1.1312x over previous
import functools

import jax
import jax.numpy as jnp
from jax import lax
from jax.experimental import pallas as pl
from jax.experimental.pallas import tpu as pltpu
from jax.experimental.pallas import tpu_sc as plsc

N = 10000
E = 160000
D = 300
H = D // 2
HP = 160
NUM_LAYERS = 5
G = 64
OUT = 64
NT = 16
NC = 2
EPS = 1e-5

ES_PER_TILE = E // (NC * NT)
ES_CH = 40
ES_STEPS = ES_PER_TILE // ES_CH
AG_PER_TILE = E // NT
AG_CH = 80
AG_STEPS = AG_PER_TILE // AG_CH
ZROWS = 1000

def _sc_edge_scalars_body(dst_hbm, payload_hbm, zrows_hbm, out0, out1,
                          spmem, idx_v, rows_v, sem):
    c = lax.axis_index("c")
    sid = lax.axis_index("s")

    @pl.when(sid < 10)
    def _():
        pltpu.sync_copy(zrows_hbm, spmem.at[pl.ds(sid * ZROWS, ZROWS)])

    plsc.subcore_barrier()

    base = (c * NT + sid) * ES_PER_TILE

    def body(i, carry):
        off = pl.multiple_of(base + i * ES_CH, 8)
        pltpu.sync_copy(dst_hbm.at[pl.ds(off, ES_CH)], idx_v)
        pltpu.async_copy(payload_hbm.at[pl.ds(off, ES_CH)], rows_v, sem).wait()
        pltpu.sync_copy(rows_v, spmem.at[idx_v], add=True)
        return carry

    lax.fori_loop(0, ES_STEPS, body, 0)
    plsc.subcore_barrier()

    @pl.when(jnp.logical_and(c == 0, sid < 10))
    def _():
        sl = pl.ds(sid * ZROWS, ZROWS)
        pltpu.sync_copy(spmem.at[sl], out0.at[sl])

    @pl.when(jnp.logical_and(c == 1, sid < 10))
    def _():
        sl = pl.ds(sid * ZROWS, ZROWS)
        pltpu.sync_copy(spmem.at[sl], out1.at[sl])


def _sc_aggregate_body(t0_hbm, t1_hbm, src_hbm, dst_hbm, zrows_hbm, out0, out1,
                       spmem, sidx_a, sidx_b, didx_a, didx_b, rows_a, rows_b,
                       sem_sa, sem_sb, sem_da, sem_db, sem_ra, sem_rb):
    c = lax.axis_index("c")
    sid = lax.axis_index("s")

    @pl.when(sid < 10)
    def _():
        pltpu.sync_copy(zrows_hbm, spmem.at[pl.ds(sid * ZROWS, ZROWS)])

    plsc.subcore_barrier()

    base = sid * AG_PER_TILE

    def run(tbl):
        bufs = ((sidx_a, didx_a, rows_a, sem_sa, sem_da, sem_ra),
                (sidx_b, didx_b, rows_b, sem_sb, sem_db, sem_rb))

        def load_idx(i, b):
            off = pl.multiple_of(base + i * AG_CH, 8)
            pltpu.async_copy(src_hbm.at[pl.ds(off, AG_CH)], b[0], b[3])
            pltpu.async_copy(dst_hbm.at[pl.ds(off, AG_CH)], b[1], b[4])

        def start_gather(b):
            pltpu.make_async_copy(src_hbm.at[pl.ds(0, AG_CH)], b[0], b[3]).wait()
            pltpu.async_copy(tbl.at[b[0]], b[2], b[5])

        def finish(b):
            pltpu.make_async_copy(tbl.at[b[0]], b[2], b[5]).wait()
            pltpu.make_async_copy(dst_hbm.at[pl.ds(0, AG_CH)], b[1], b[4]).wait()
            pltpu.sync_copy(b[2], spmem.at[b[1]], add=True)

        load_idx(0, bufs[0])
        start_gather(bufs[0])
        load_idx(1, bufs[1])

        def step(i, cur, nxt):
            @pl.when(i + 1 < AG_STEPS)
            def _():
                start_gather(nxt)
            finish(cur)
            @pl.when(i + 2 < AG_STEPS)
            def _():
                load_idx(i + 2, cur)

        def body(i, carry):
            @pl.when(i % 2 == 0)
            def _():
                step(i, bufs[0], bufs[1])
            @pl.when(i % 2 == 1)
            def _():
                step(i, bufs[1], bufs[0])
            return carry

        lax.fori_loop(0, AG_STEPS, body, 0)

    @pl.when(c == 0)
    def _():
        run(t0_hbm)

    @pl.when(c == 1)
    def _():
        run(t1_hbm)

    plsc.subcore_barrier()

    @pl.when(jnp.logical_and(c == 0, sid < 10))
    def _():
        sl = pl.ds(sid * ZROWS, ZROWS)
        pltpu.sync_copy(spmem.at[sl], out0.at[sl])

    @pl.when(jnp.logical_and(c == 1, sid < 10))
    def _():
        sl = pl.ds(sid * ZROWS, ZROWS)
        pltpu.sync_copy(spmem.at[sl], out1.at[sl])


@functools.cache
def _build_sc_kernels():
    mesh = plsc.VectorSubcoreMesh(core_axis_name="c", subcore_axis_name="s",
                                  num_cores=NC, num_subcores=NT)
    sc_edge_scalars = pl.kernel(
        _sc_edge_scalars_body,
        out_type=[
            jax.ShapeDtypeStruct((N, 16), jnp.float32),
            jax.ShapeDtypeStruct((N, 16), jnp.float32),
        ],
        mesh=mesh,
        scratch_types=[
            pltpu.VMEM_SHARED((N, 16), jnp.float32),
            pltpu.VMEM((ES_CH,), jnp.int32),
            pltpu.VMEM((ES_CH, 16), jnp.float32),
            pltpu.SemaphoreType.DMA,
        ],
        compiler_params=pltpu.CompilerParams(use_tc_tiling_on_sc=False),
    )
    sc_aggregate = pl.kernel(
        _sc_aggregate_body,
        out_type=[
            jax.ShapeDtypeStruct((N, HP), jnp.float32),
            jax.ShapeDtypeStruct((N, HP), jnp.float32),
        ],
        mesh=mesh,
        scratch_types=[
            pltpu.VMEM_SHARED((N, HP), jnp.float32),
            pltpu.VMEM((AG_CH,), jnp.int32),
            pltpu.VMEM((AG_CH,), jnp.int32),
            pltpu.VMEM((AG_CH,), jnp.int32),
            pltpu.VMEM((AG_CH,), jnp.int32),
            pltpu.VMEM((AG_CH, HP), jnp.float32),
            pltpu.VMEM((AG_CH, HP), jnp.float32),
            pltpu.SemaphoreType.DMA,
            pltpu.SemaphoreType.DMA,
            pltpu.SemaphoreType.DMA,
            pltpu.SemaphoreType.DMA,
            pltpu.SemaphoreType.DMA,
            pltpu.SemaphoreType.DMA,
        ],
        compiler_params=pltpu.CompilerParams(use_tc_tiling_on_sc=False),
    )
    return sc_edge_scalars, sc_aggregate


RB = 1000
NRB = N // RB


def _tc_mlp_body(a0_ref, a1_ref, es0_ref, es1_ref, stats_ref, gamma_ref,
                 beta_ref, w1_ref, b1_ref, w2_ref, b2_ref,
                 z0_ref, z1_ref, ostats_ref):
    i = pl.program_id(0)
    mean = stats_ref[0:1, :] * (1.0 / N)
    var = stats_ref[1:2, :] * (1.0 / N) - mean * mean
    s = gamma_ref[...] * lax.rsqrt(var + EPS)
    t = beta_ref[...] - mean * s

    esum = es0_ref[:, 0:1] + es1_ref[:, 0:1]
    indeg = es0_ref[:, 1:2] + es1_ref[:, 1:2]

    a = jnp.concatenate([a0_ref[...][:, :H], a1_ref[...][:, :H]], axis=1)
    agg = a * s + indeg * t + esum
    pre = jnp.dot(agg, w1_ref[...], preferred_element_type=jnp.float32)
    pre += b1_ref[...]
    h1 = jnp.maximum(pre, 0.0)
    z = jnp.dot(h1, w2_ref[...], preferred_element_type=jnp.float32)
    z += b2_ref[...]

    zpad = jnp.zeros((RB, HP - H), jnp.float32)
    z0_ref[...] = jnp.concatenate([z[:, :H], zpad], axis=1)
    z1_ref[...] = jnp.concatenate([z[:, H:], zpad], axis=1)

    @pl.when(i == 0)
    def _():
        ostats_ref[...] = jnp.zeros_like(ostats_ref)

    ostats_ref[0:1, :] += jnp.sum(z, axis=0, keepdims=True)
    ostats_ref[1:2, :] += jnp.sum(z * z, axis=0, keepdims=True)


_tc_mlp = pl.pallas_call(
    _tc_mlp_body,
    grid=(NRB,),
    in_specs=[
        pl.BlockSpec((RB, HP), lambda i: (i, 0)),
        pl.BlockSpec((RB, HP), lambda i: (i, 0)),
        pl.BlockSpec((RB, 16), lambda i: (i, 0)),
        pl.BlockSpec((RB, 16), lambda i: (i, 0)),
        pl.BlockSpec((2, D), lambda i: (0, 0)),
        pl.BlockSpec((1, D), lambda i: (0, 0)),
        pl.BlockSpec((1, D), lambda i: (0, 0)),
        pl.BlockSpec((D, 2 * D), lambda i: (0, 0)),
        pl.BlockSpec((1, 2 * D), lambda i: (0, 0)),
        pl.BlockSpec((2 * D, D), lambda i: (0, 0)),
        pl.BlockSpec((1, D), lambda i: (0, 0)),
    ],
    out_specs=[
        pl.BlockSpec((RB, HP), lambda i: (i, 0)),
        pl.BlockSpec((RB, HP), lambda i: (i, 0)),
        pl.BlockSpec((2, D), lambda i: (0, 0)),
    ],
    out_shape=[
        jax.ShapeDtypeStruct((N, HP), jnp.float32),
        jax.ShapeDtypeStruct((N, HP), jnp.float32),
        jax.ShapeDtypeStruct((2, D), jnp.float32),
    ],
)


def _tc_readout_body(z0_ref, z1_ref, gids_ref, stats_ref, gamma_ref, beta_ref,
                     wt_ref, bt_ref, out_ref, acc0, acc1, cnt):
    i = pl.program_id(0)

    @pl.when(i == 0)
    def _():
        acc0[...] = jnp.zeros_like(acc0)
        acc1[...] = jnp.zeros_like(acc1)
        cnt[...] = jnp.zeros_like(cnt)

    gids = gids_ref[...]
    giota = lax.broadcasted_iota(jnp.int32, (RB, G), 1).astype(jnp.float32)
    m = (giota == gids).astype(jnp.float32)
    dn = (((0,), (0,)), ((), ()))
    acc0[...] += lax.dot_general(m, z0_ref[...][:, :H], dn,
                                 preferred_element_type=jnp.float32,
                  precision=lax.Precision.HIGHEST)
    acc1[...] += lax.dot_general(m, z1_ref[...][:, :H], dn,
                                 preferred_element_type=jnp.float32,
                  precision=lax.Precision.HIGHEST)
    ones = jnp.ones((RB, 1), jnp.float32)
    cnt[...] += lax.dot_general(m, ones, dn,
                                preferred_element_type=jnp.float32,
                  precision=lax.Precision.HIGHEST)

    @pl.when(i == NRB - 1)
    def _():
        mean = stats_ref[0:1, :] * (1.0 / N)
        var = stats_ref[1:2, :] * (1.0 / N) - mean * mean
        s = gamma_ref[...] * lax.rsqrt(var + EPS)
        t = beta_ref[...] - mean * s
        c = cnt[...]
        inv = 1.0 / jnp.maximum(c, 1.0)
        g0 = (acc0[...] * s[:, :H] + c * t[:, :H]) * inv
        g1 = (acc1[...] * s[:, H:] + c * t[:, H:]) * inv
        gm = jnp.concatenate([g0, g1], axis=1)
        o = jnp.dot(gm, wt_ref[...], preferred_element_type=jnp.float32)
        out_ref[...] = o + bt_ref[...]


_tc_readout = pl.pallas_call(
    _tc_readout_body,
    grid=(NRB,),
    in_specs=[
        pl.BlockSpec((RB, HP), lambda i: (i, 0)),
        pl.BlockSpec((RB, HP), lambda i: (i, 0)),
        pl.BlockSpec((RB, 1), lambda i: (i, 0)),
        pl.BlockSpec((2, D), lambda i: (0, 0)),
        pl.BlockSpec((1, D), lambda i: (0, 0)),
        pl.BlockSpec((1, D), lambda i: (0, 0)),
        pl.BlockSpec((D, OUT), lambda i: (0, 0)),
        pl.BlockSpec((1, OUT), lambda i: (0, 0)),
    ],
    out_specs=pl.BlockSpec((G, OUT), lambda i: (0, 0)),
    out_shape=jax.ShapeDtypeStruct((G, OUT), jnp.float32),
    scratch_shapes=[
        pltpu.VMEM((G, H), jnp.float32),
        pltpu.VMEM((G, H), jnp.float32),
        pltpu.VMEM((G, 1), jnp.float32),
    ],
)


def kernel(node_feats, edge_feats, edge_index, graph_ids, params, Wt, bt):
    src = edge_index[0]
    dst = edge_index[1]

    nf0 = jnp.pad(node_feats[:, :H], ((0, 0), (0, HP - H)))
    nf1 = jnp.pad(node_feats[:, H:], ((0, 0), (0, HP - H)))
    ones_e = jnp.ones((E,), jnp.float32)
    payload = jnp.tile(jnp.stack([edge_feats[:, 0], ones_e], axis=-1), (1, 8))
    zrows16 = jnp.zeros((ZROWS, 16), jnp.float32)
    zrowsH = jnp.zeros((ZROWS, HP), jnp.float32)
    gids = graph_ids.astype(jnp.float32).reshape(N, 1)

    sc_edge_scalars, sc_aggregate = _build_sc_kernels()
    es0, es1 = sc_edge_scalars(dst, payload, zrows16)

    stats = jnp.stack([jnp.zeros((D,), jnp.float32),
                       jnp.full((D,), N * (1.0 - EPS), jnp.float32)])
    gamma_p = jnp.ones((1, D), jnp.float32)
    beta_p = jnp.zeros((1, D), jnp.float32)

    h0, h1 = nf0, nf1
    for (W1, b1, W2, b2, gamma, beta) in params:
        a0, a1 = sc_aggregate(h0, h1, src, dst, zrowsH)
        h0, h1, stats = _tc_mlp(a0, a1, es0, es1, stats, gamma_p, beta_p,
                                W1, b1.reshape(1, 2 * D), W2,
                                b2.reshape(1, D))
        gamma_p = gamma.reshape(1, D)
        beta_p = beta.reshape(1, D)

    return _tc_readout(h0, h1, gids, stats, gamma_p, beta_p,
                       Wt, bt.reshape(1, OUT))
